# baseline (speedup 1.0000x reference)
import functools

import jax
import jax.numpy as jnp
from jax import lax
from jax.experimental import pallas as pl
from jax.experimental.pallas import tpu as pltpu
from jax.experimental.pallas import tpu_sc as plsc

D = 64
C = 200
NW = 32
EPS = 1e-12


def _rsqrt(t):
    i = lax.bitcast_convert_type(t, jnp.int32)
    i = jnp.int32(0x5F3759DF) - lax.shift_right_arithmetic(i, 1)
    y = lax.bitcast_convert_type(i, jnp.float32)
    for _ in range(3):
        y = y * (1.5 - 0.5 * t * y * y)
    return y


@functools.cache
def _build(B, S, V, PT):
    N = B * S
    n_chunks = N // (NW * C)
    n_pairs = n_chunks // 2
    mesh = plsc.VectorSubcoreMesh(core_axis_name="c", subcore_axis_name="s")

    @functools.partial(
        pl.kernel,
        mesh=mesh,
        out_type=jax.ShapeDtypeStruct((B, S, D), jnp.float32),
        compiler_params=pltpu.CompilerParams(
            needs_layout_passes=False, use_tc_tiling_on_sc=False
        ),
        scratch_types=[
            pltpu.VMEM((C, D), jnp.float32),
            pltpu.VMEM((C, D), jnp.float32),
            pltpu.VMEM((C, D), jnp.float32),
            pltpu.VMEM((C, D), jnp.float32),
            pltpu.VMEM((C, D), jnp.float32),
            pltpu.VMEM((C, D), jnp.float32),
            pltpu.VMEM((C,), jnp.int32),
            pltpu.VMEM((C,), jnp.int32),
            pltpu.VMEM((C,), jnp.int32),
            pltpu.VMEM((C,), jnp.int32),
            pltpu.VMEM((D,), jnp.float32),
            pltpu.VMEM((D,), jnp.float32),
            pltpu.SemaphoreType.DMA,
            pltpu.SemaphoreType.DMA,
            pltpu.SemaphoreType.DMA,
            pltpu.SemaphoreType.DMA,
        ],
    )
    def body(ids_h, ptid_h, word_h, pt_h, gam_h, bet_h, out_h,
             ws0, ws1, ps0, ps1, os0, os1, iw0, iw1, ip0, ip1,
             gam_v, bet_v, gsem0, gsem1, osem0, osem1):
        wid = lax.axis_index("s") * 2 + lax.axis_index("c")
        tok0 = wid * (n_chunks * C)

        pltpu.sync_copy(gam_h, gam_v)
        pltpu.sync_copy(bet_h, bet_v)
        gk = [gam_v[pl.ds(k * 16, 16)] for k in range(4)]
        bk = [bet_v[pl.ds(k * 16, 16)] for k in range(4)]

        def issue_gathers(base, iw, ip, ws, ps, gsem):
            pltpu.sync_copy(ids_h.at[pl.ds(base, C)], iw)
            pltpu.sync_copy(ptid_h.at[pl.ds(base, C)], ip)
            for lo, n in ((0, 128), (128, C - 128)):
                sl = pl.ds(lo, n)
                pltpu.async_copy(word_h.at[iw.at[sl]], ws.at[sl], gsem)
                pltpu.async_copy(pt_h.at[ip.at[sl]], ps.at[sl], gsem)

        def drain_gathers(ws, ps, gsem):
            pltpu.make_async_copy(word_h.at[pl.ds(0, C)], ws, gsem).wait()
            pltpu.make_async_copy(pt_h.at[pl.ds(0, C)], ps, gsem).wait()

        def drain_out(os_, row, osem):
            pltpu.make_async_copy(os_, out_h.at[row], osem).wait()

        def compute(ws, ps, os_):
            def tok2(t2, carry):
                for tt in range(2):
                    t = t2 * 2 + tt
                    xs = []
                    for k in range(4):
                        sl = pl.ds(k * 16, 16)
                        xs.append(ws[t, sl] + ps[t, sl])
                    sv = (xs[0] + xs[1]) + (xs[2] + xs[3])
                    qv = ((xs[0] * xs[0] + xs[1] * xs[1])
                          + (xs[2] * xs[2] + xs[3] * xs[3]))
                    mean = jnp.sum(sv) * (1.0 / D)
                    var = jnp.sum(qv) * (1.0 / D) - mean * mean
                    rstd = _rsqrt(var + EPS)
                    for k in range(4):
                        sl = pl.ds(k * 16, 16)
                        os_[t, sl] = (xs[k] - mean) * rstd * gk[k] + bk[k]
                return carry

            lax.fori_loop(0, C // 2, tok2, 0)

        row0 = wid * n_chunks

        issue_gathers(tok0, iw0, ip0, ws0, ps0, gsem0)

        def pair_body(p, carry):
            base0 = tok0 + (2 * p) * C
            base1 = base0 + C
            r0 = row0 + 2 * p
            issue_gathers(base1, iw1, ip1, ws1, ps1, gsem1)
            drain_gathers(ws0, ps0, gsem0)

            @pl.when(p > 0)
            def _():
                drain_out(os0, r0, osem0)

            compute(ws0, ps0, os0)
            pltpu.async_copy(os0, out_h.at[r0], osem0)

            @pl.when(p < n_pairs - 1)
            def _():
                issue_gathers(base1 + C, iw0, ip0, ws0, ps0, gsem0)

            drain_gathers(ws1, ps1, gsem1)

            @pl.when(p > 0)
            def _():
                drain_out(os1, r0 + 1, osem1)

            compute(ws1, ps1, os1)
            pltpu.async_copy(os1, out_h.at[r0 + 1], osem1)
            return carry

        lax.fori_loop(0, n_pairs, pair_body, 0)
        drain_out(os0, row0, osem0)
        drain_out(os1, row0, osem1)

    return body


def kernel(input_ids, token_type_ids, position_ids, word_emb, pos_emb,
           type_emb, ln_gamma, ln_beta):
    B, S = input_ids.shape
    N = B * S
    V, _ = word_emb.shape
    P, _ = pos_emb.shape
    T, _ = type_emb.shape
    pt_tab = (pos_emb[:, None, :] + type_emb[None, :, :]).reshape(P * T, D)
    ptid = (position_ids.astype(jnp.int32) * T
            + token_type_ids.astype(jnp.int32)).reshape(N)
    body = _build(B, S, V, P * T)
    return body(
        input_ids.reshape(N).astype(jnp.int32),
        ptid,
        word_emb,
        pt_tab,
        ln_gamma,
        ln_beta,
    )


# speedup vs baseline: 1.0028x; 1.0028x over previous
import functools

import jax
import jax.numpy as jnp
from jax import lax
from jax.experimental import pallas as pl
from jax.experimental.pallas import tpu as pltpu
from jax.experimental.pallas import tpu_sc as plsc

D = 64
C = 200
NW = 32
EPS = 1e-12


def _rsqrt(t):
    i = lax.bitcast_convert_type(t, jnp.int32)
    i = jnp.int32(0x5F3759DF) - lax.shift_right_arithmetic(i, 1)
    y = lax.bitcast_convert_type(i, jnp.float32)
    for _ in range(3):
        y = y * (1.5 - 0.5 * t * y * y)
    return y


@functools.cache
def _build(B, S, V, PT):
    N = B * S
    n_chunks = N // (NW * C)
    n_pairs = n_chunks // 2
    mesh = plsc.VectorSubcoreMesh(core_axis_name="c", subcore_axis_name="s")

    @functools.partial(
        pl.kernel,
        mesh=mesh,
        out_type=jax.ShapeDtypeStruct((B, S, D), jnp.float32),
        compiler_params=pltpu.CompilerParams(
            needs_layout_passes=False, use_tc_tiling_on_sc=False
        ),
        scratch_types=[
            pltpu.VMEM((C, D), jnp.float32),
            pltpu.VMEM((C, D), jnp.float32),
            pltpu.VMEM((C, D), jnp.float32),
            pltpu.VMEM((C, D), jnp.float32),
            pltpu.VMEM((C, D), jnp.float32),
            pltpu.VMEM((C, D), jnp.float32),
            pltpu.VMEM((C,), jnp.int32),
            pltpu.VMEM((C,), jnp.int32),
            pltpu.VMEM((C,), jnp.int32),
            pltpu.VMEM((C,), jnp.int32),
            pltpu.VMEM((D,), jnp.float32),
            pltpu.VMEM((D,), jnp.float32),
            pltpu.SemaphoreType.DMA,
            pltpu.SemaphoreType.DMA,
            pltpu.SemaphoreType.DMA,
            pltpu.SemaphoreType.DMA,
        ],
    )
    def body(ids_h, ptid_h, word_h, pt_h, gam_h, bet_h, out_h,
             ws0, ws1, ps0, ps1, os0, os1, iw0, iw1, ip0, ip1,
             gam_v, bet_v, gsem0, gsem1, osem0, osem1):
        wid = lax.axis_index("s") * 2 + lax.axis_index("c")
        tok0 = wid * (n_chunks * C)

        pltpu.sync_copy(gam_h, gam_v)
        pltpu.sync_copy(bet_h, bet_v)
        gk = [gam_v[pl.ds(k * 16, 16)] for k in range(4)]
        bk = [bet_v[pl.ds(k * 16, 16)] for k in range(4)]

        def issue_gathers(base, iw, ip, ws, ps, gsem):
            pltpu.sync_copy(ids_h.at[pl.ds(base, C)], iw)
            pltpu.sync_copy(ptid_h.at[pl.ds(base, C)], ip)
            for lo, n in ((0, 128), (128, C - 128)):
                sl = pl.ds(lo, n)
                pltpu.async_copy(word_h.at[iw.at[sl]], ws.at[sl], gsem)
                pltpu.async_copy(pt_h.at[ip.at[sl]], ps.at[sl], gsem)

        def drain_gathers(ws, ps, gsem):
            pltpu.make_async_copy(word_h.at[pl.ds(0, C)], ws, gsem).wait()
            pltpu.make_async_copy(pt_h.at[pl.ds(0, C)], ps, gsem).wait()

        def drain_out(os_, row, osem):
            pltpu.make_async_copy(os_, out_h.at[row], osem).wait()

        def compute(ws, ps, os_):
            def tok2(t2, carry):
                for tt in range(4):
                    t = t2 * 4 + tt
                    xs = []
                    for k in range(4):
                        sl = pl.ds(k * 16, 16)
                        xs.append(ws[t, sl] + ps[t, sl])
                    sv = (xs[0] + xs[1]) + (xs[2] + xs[3])
                    qv = ((xs[0] * xs[0] + xs[1] * xs[1])
                          + (xs[2] * xs[2] + xs[3] * xs[3]))
                    mean = jnp.sum(sv) * (1.0 / D)
                    var = jnp.sum(qv) * (1.0 / D) - mean * mean
                    rstd = _rsqrt(var + EPS)
                    for k in range(4):
                        sl = pl.ds(k * 16, 16)
                        os_[t, sl] = (xs[k] - mean) * rstd * gk[k] + bk[k]
                return carry

            lax.fori_loop(0, C // 4, tok2, 0)

        row0 = wid * n_chunks

        issue_gathers(tok0, iw0, ip0, ws0, ps0, gsem0)

        def pair_body(p, carry):
            base0 = tok0 + (2 * p) * C
            base1 = base0 + C
            r0 = row0 + 2 * p
            issue_gathers(base1, iw1, ip1, ws1, ps1, gsem1)
            drain_gathers(ws0, ps0, gsem0)

            @pl.when(p > 0)
            def _():
                drain_out(os0, r0, osem0)

            compute(ws0, ps0, os0)
            pltpu.async_copy(os0, out_h.at[r0], osem0)

            @pl.when(p < n_pairs - 1)
            def _():
                issue_gathers(base1 + C, iw0, ip0, ws0, ps0, gsem0)

            drain_gathers(ws1, ps1, gsem1)

            @pl.when(p > 0)
            def _():
                drain_out(os1, r0 + 1, osem1)

            compute(ws1, ps1, os1)
            pltpu.async_copy(os1, out_h.at[r0 + 1], osem1)
            return carry

        lax.fori_loop(0, n_pairs, pair_body, 0)
        drain_out(os0, row0, osem0)
        drain_out(os1, row0, osem1)

    return body


def kernel(input_ids, token_type_ids, position_ids, word_emb, pos_emb,
           type_emb, ln_gamma, ln_beta):
    B, S = input_ids.shape
    N = B * S
    V, _ = word_emb.shape
    P, _ = pos_emb.shape
    T, _ = type_emb.shape
    pt_tab = (pos_emb[:, None, :] + type_emb[None, :, :]).reshape(P * T, D)
    ptid = (position_ids.astype(jnp.int32) * T
            + token_type_ids.astype(jnp.int32)).reshape(N)
    body = _build(B, S, V, P * T)
    return body(
        input_ids.reshape(N).astype(jnp.int32),
        ptid,
        word_emb * jnp.float32(1.0),
        pt_tab,
        ln_gamma,
        ln_beta,
    )


# speedup vs baseline: 1.0244x; 1.0216x over previous
import functools

import jax
import jax.numpy as jnp
from jax import lax
from jax.experimental import pallas as pl
from jax.experimental.pallas import tpu as pltpu
from jax.experimental.pallas import tpu_sc as plsc

D = 64
C = 256
NW = 32
EPS = 1e-12


def _rsqrt(t):
    i = lax.bitcast_convert_type(t, jnp.int32)
    i = jnp.int32(0x5F3759DF) - lax.shift_right_arithmetic(i, 1)
    y = lax.bitcast_convert_type(i, jnp.float32)
    for _ in range(3):
        y = y * (1.5 - 0.5 * t * y * y)
    return y


@functools.cache
def _build(N, V, PT):
    n_chunks = N // (NW * C)
    n_pairs = n_chunks // 2
    mesh = plsc.VectorSubcoreMesh(core_axis_name="c", subcore_axis_name="s")

    @functools.partial(
        pl.kernel,
        mesh=mesh,
        out_type=jax.ShapeDtypeStruct((N, D), jnp.float32),
        compiler_params=pltpu.CompilerParams(
            needs_layout_passes=False, use_tc_tiling_on_sc=False
        ),
        scratch_types=[
            pltpu.VMEM((C, D), jnp.float32),
            pltpu.VMEM((C, D), jnp.float32),
            pltpu.VMEM((C, D), jnp.float32),
            pltpu.VMEM((C, D), jnp.float32),
            pltpu.VMEM((C, D), jnp.float32),
            pltpu.VMEM((C, D), jnp.float32),
            pltpu.VMEM((C,), jnp.int32),
            pltpu.VMEM((C,), jnp.int32),
            pltpu.VMEM((C,), jnp.int32),
            pltpu.VMEM((C,), jnp.int32),
            pltpu.VMEM((D,), jnp.float32),
            pltpu.VMEM((D,), jnp.float32),
            pltpu.SemaphoreType.DMA,
            pltpu.SemaphoreType.DMA,
            pltpu.SemaphoreType.DMA,
            pltpu.SemaphoreType.DMA,
        ],
    )
    def body(ids_h, ptid_h, word_h, pt_h, gam_h, bet_h, out_h,
             ws0, ws1, ps0, ps1, os0, os1, iw0, iw1, ip0, ip1,
             gam_v, bet_v, gsem0, gsem1, osem0, osem1):
        wid = lax.axis_index("s") * 2 + lax.axis_index("c")
        tok0 = wid * (n_chunks * C)

        pltpu.sync_copy(gam_h, gam_v)
        pltpu.sync_copy(bet_h, bet_v)
        gk = [gam_v[pl.ds(k * 16, 16)] for k in range(4)]
        bk = [bet_v[pl.ds(k * 16, 16)] for k in range(4)]

        def issue_gathers(base, iw, ip, ws, ps, gsem):
            pltpu.sync_copy(ids_h.at[pl.ds(base, C)], iw)
            pltpu.sync_copy(ptid_h.at[pl.ds(base, C)], ip)
            for j in range(C // 128):
                sl = pl.ds(j * 128, 128)
                pltpu.async_copy(word_h.at[iw.at[sl]], ws.at[sl], gsem)
                pltpu.async_copy(pt_h.at[ip.at[sl]], ps.at[sl], gsem)

        def drain_gathers(ws, ps, gsem):
            pltpu.make_async_copy(word_h.at[pl.ds(0, C)], ws, gsem).wait()
            pltpu.make_async_copy(pt_h.at[pl.ds(0, C)], ps, gsem).wait()

        def drain_out(os_, base, osem):
            pltpu.make_async_copy(os_, out_h.at[pl.ds(base, C)], osem).wait()

        def compute(ws, ps, os_):
            def tok2(t2, carry):
                for tt in range(2):
                    t = t2 * 2 + tt
                    xs = []
                    for k in range(4):
                        sl = pl.ds(k * 16, 16)
                        xs.append(ws[t, sl] + ps[t, sl])
                    sv = (xs[0] + xs[1]) + (xs[2] + xs[3])
                    qv = ((xs[0] * xs[0] + xs[1] * xs[1])
                          + (xs[2] * xs[2] + xs[3] * xs[3]))
                    mean = jnp.sum(sv) * (1.0 / D)
                    var = jnp.sum(qv) * (1.0 / D) - mean * mean
                    rstd = _rsqrt(var + EPS)
                    for k in range(4):
                        sl = pl.ds(k * 16, 16)
                        os_[t, sl] = (xs[k] - mean) * rstd * gk[k] + bk[k]
                return carry

            lax.fori_loop(0, C // 2, tok2, 0)

        issue_gathers(tok0, iw0, ip0, ws0, ps0, gsem0)

        def pair_body(p, carry):
            base0 = tok0 + (2 * p) * C
            base1 = base0 + C
            issue_gathers(base1, iw1, ip1, ws1, ps1, gsem1)
            drain_gathers(ws0, ps0, gsem0)

            @pl.when(p > 0)
            def _():
                drain_out(os0, base0, osem0)

            compute(ws0, ps0, os0)
            pltpu.async_copy(os0, out_h.at[pl.ds(base0, C)], osem0)

            @pl.when(p < n_pairs - 1)
            def _():
                issue_gathers(base1 + C, iw0, ip0, ws0, ps0, gsem0)

            drain_gathers(ws1, ps1, gsem1)

            @pl.when(p > 0)
            def _():
                drain_out(os1, base1, osem1)

            compute(ws1, ps1, os1)
            pltpu.async_copy(os1, out_h.at[pl.ds(base1, C)], osem1)
            return carry

        lax.fori_loop(0, n_pairs, pair_body, 0)
        drain_out(os0, tok0, osem0)
        drain_out(os1, tok0, osem1)

    return body


def kernel(input_ids, token_type_ids, position_ids, word_emb, pos_emb,
           type_emb, ln_gamma, ln_beta):
    B, S = input_ids.shape
    N = B * S
    V, _ = word_emb.shape
    P, _ = pos_emb.shape
    T, _ = type_emb.shape
    pt_tab = (pos_emb[:, None, :] + type_emb[None, :, :]).reshape(P * T, D)
    ptid = (position_ids.astype(jnp.int32) * T
            + token_type_ids.astype(jnp.int32)).reshape(N)
    body = _build(N, V, P * T)
    out = body(
        input_ids.reshape(N).astype(jnp.int32),
        ptid,
        word_emb,
        pt_tab,
        ln_gamma,
        ln_beta,
    )
    return out.reshape(B, S, D)
